# baseline (speedup 1.0000x reference)
import functools
import math

import jax
import jax.numpy as jnp
from jax.experimental import pallas as pl
from jax.experimental.pallas import tpu as pltpu

T = 8
D = 1024
H = 512
NS = 2
N = 2048
TILE = 512
DH = D // 2
LOG2PI = math.log(2.0 * math.pi)


def _flow_kernel(x_ref, pi_ref, mus_ref, ls_ref,
                 Ws1_ref, bs1_ref, Ws2_ref, bs2_ref,
                 Wc1_ref, bc1_ref, Wc2_ref, bc2_ref,
                 out_ref,
                 A_ref, B_ref, lds_ref, lk_ref):
    k = pl.program_id(1)
    x1 = x_ref[:, :DH]
    x1b = x1.astype(jnp.bfloat16)

    @pl.when(k == 0)
    def _shared():
        h1 = jnp.tanh(jnp.dot(x1b, Ws1_ref[1], preferred_element_type=jnp.float32)
                      + bs1_ref[1][None, :])
        st1 = jnp.dot(h1.astype(jnp.bfloat16), Ws2_ref[1],
                      preferred_element_type=jnp.float32) + bs2_ref[1][None, :]
        s1 = jnp.tanh(st1[:, :DH])
        t1 = st1[:, DH:]
        h0 = jnp.tanh(jnp.dot(x1b, Ws1_ref[0], preferred_element_type=jnp.float32)
                      + bs1_ref[0][None, :])
        st0 = jnp.dot(h0.astype(jnp.bfloat16), Ws2_ref[0],
                      preferred_element_type=jnp.float32) + bs2_ref[0][None, :]
        s0 = jnp.tanh(st0[:, :DH])
        t0 = st0[:, DH:]
        e0 = jnp.exp(-s0)
        A = jnp.exp(-(s0 + s1))
        A_ref[:, :] = A
        B_ref[:, :] = -(t1 * A + t0 * e0)
        lds_ref[0, :] = -jnp.sum(s0 + s1, axis=1)

    h = jnp.tanh(jnp.dot(x1b, Wc1_ref[0], preferred_element_type=jnp.float32)
                 + bc1_ref[0])
    st = jnp.dot(h.astype(jnp.bfloat16), Wc2_ref[0],
                 preferred_element_type=jnp.float32) + bc2_ref[0]
    s = jnp.tanh(st[:, :DH])
    t = st[:, DH:]
    x2 = x_ref[:, DH:]
    z2 = (x2 - t) * jnp.exp(-s) * A_ref[:, :] + B_ref[:, :]

    mu = mus_ref[0, 0]
    ls = ls_ref[0, 0]
    r1 = (x1 - mu[None, :DH]) * jnp.exp(-ls[None, :DH])
    r2 = (z2 - mu[None, DH:]) * jnp.exp(-ls[None, DH:])
    g = -0.5 * (jnp.sum(r1 * r1, axis=1) + jnp.sum(r2 * r2, axis=1)
                + 2.0 * jnp.sum(ls) + D * LOG2PI)
    lk_ref[pl.ds(k, 1), :] = (g - jnp.sum(s, axis=1))[None, :]

    @pl.when(k == T - 1)
    def _finish():
        pi = pi_ref[0, :]
        la = pi - (jnp.max(pi) + jnp.log(jnp.sum(jnp.exp(pi - jnp.max(pi)))))
        lp = lk_ref[:, :] + la[:, None]
        m = jnp.max(lp, axis=0)
        lse = m + jnp.log(jnp.sum(jnp.exp(lp - m[None, :]), axis=0))
        out_ref[0, :] = lse + lds_ref[0, :]


@jax.jit
def kernel(x, pi_logits, mus, log_sigmas, Ws1, bs1, Ws2, bs2, Wc1, bc1, Wc2, bc2):
    n = x.shape[0]
    Ws1 = Ws1.astype(jnp.bfloat16)
    Ws2 = Ws2.astype(jnp.bfloat16)
    Wc1 = Wc1.astype(jnp.bfloat16)
    Wc2 = Wc2.astype(jnp.bfloat16)
    Wc1r = Wc1.reshape(T, DH, H)
    bc1r = bc1.reshape(T, 1, H)
    Wc2r = Wc2.reshape(T, H, D)
    bc2r = bc2.reshape(T, 1, D)
    mus3 = mus.reshape(T, 1, D)
    ls3 = log_sigmas.reshape(T, 1, D)
    pi2 = pi_logits.reshape(1, T)

    grid = (n // TILE, T)
    out = pl.pallas_call(
        _flow_kernel,
        grid=grid,
        in_specs=[
            pl.BlockSpec((TILE, D), lambda i, k: (i, 0)),
            pl.BlockSpec((1, T), lambda i, k: (0, 0)),
            pl.BlockSpec((1, 1, D), lambda i, k: (k, 0, 0)),
            pl.BlockSpec((1, 1, D), lambda i, k: (k, 0, 0)),
            pl.BlockSpec((NS, DH, H), lambda i, k: (0, 0, 0)),
            pl.BlockSpec((NS, H), lambda i, k: (0, 0)),
            pl.BlockSpec((NS, H, D), lambda i, k: (0, 0, 0)),
            pl.BlockSpec((NS, D), lambda i, k: (0, 0)),
            pl.BlockSpec((1, DH, H), lambda i, k: (k, 0, 0)),
            pl.BlockSpec((1, 1, H), lambda i, k: (k, 0, 0)),
            pl.BlockSpec((1, H, D), lambda i, k: (k, 0, 0)),
            pl.BlockSpec((1, 1, D), lambda i, k: (k, 0, 0)),
        ],
        out_specs=pl.BlockSpec((1, TILE), lambda i, k: (0, i)),
        out_shape=jax.ShapeDtypeStruct((1, n), jnp.float32),
        scratch_shapes=[
            pltpu.VMEM((TILE, DH), jnp.float32),
            pltpu.VMEM((TILE, DH), jnp.float32),
            pltpu.VMEM((1, TILE), jnp.float32),
            pltpu.VMEM((T, TILE), jnp.float32),
        ],
    )(x, pi2, mus3, ls3, Ws1, bs1, Ws2, bs2, Wc1r, bc1r, Wc2r, bc2r)
    return out[0]


# speedup vs baseline: 1.2958x; 1.2958x over previous
import math

import jax
import jax.numpy as jnp
from jax.experimental import pallas as pl
from jax.experimental.pallas import tpu as pltpu

T = 8
D = 1024
H = 512
NS = 2
N = 2048
TILE = 512
KB = 2
DH = D // 2
LOG2PI = math.log(2.0 * math.pi)


def _flow_kernel(x_ref, pi_ref, mus_ref, ls_ref,
                 Ws1_ref, bs1_ref, Ws2_ref, bs2_ref,
                 Wc1_ref, bc1_ref, Wc2_ref, bc2_ref,
                 out_ref,
                 A_ref, B_ref, lds_ref, lk_ref):
    k = pl.program_id(1)
    x1 = x_ref[:, :DH]
    x2 = x_ref[:, DH:]

    @pl.when(k == 0)
    def _shared():
        h1 = jnp.tanh(jnp.dot(x1, Ws1_ref[1], preferred_element_type=jnp.float32)
                      + bs1_ref[1][None, :])
        st1 = jnp.dot(h1, Ws2_ref[1], preferred_element_type=jnp.float32) + bs2_ref[1][None, :]
        s1 = jnp.tanh(st1[:, :DH])
        t1 = st1[:, DH:]
        h0 = jnp.tanh(jnp.dot(x1, Ws1_ref[0], preferred_element_type=jnp.float32)
                      + bs1_ref[0][None, :])
        st0 = jnp.dot(h0, Ws2_ref[0], preferred_element_type=jnp.float32) + bs2_ref[0][None, :]
        s0 = jnp.tanh(st0[:, :DH])
        t0 = st0[:, DH:]
        e0 = jnp.exp(-s0)
        A = jnp.exp(-(s0 + s1))
        A_ref[:, :] = A
        B_ref[:, :] = -(t1 * A + t0 * e0)
        lds_ref[0, :] = -jnp.sum(s0 + s1, axis=1)

    for j in range(KB):
        h = jnp.tanh(jnp.dot(x1, Wc1_ref[j], preferred_element_type=jnp.float32)
                     + bc1_ref[j])
        st = jnp.dot(h, Wc2_ref[j], preferred_element_type=jnp.float32) + bc2_ref[j]
        s = jnp.tanh(st[:, :DH])
        t = st[:, DH:]
        z2 = (x2 - t) * jnp.exp(-s) * A_ref[:, :] + B_ref[:, :]

        mu = mus_ref[j, 0]
        ls = ls_ref[j, 0]
        r1 = (x1 - mu[None, :DH]) * jnp.exp(-ls[None, :DH])
        r2 = (z2 - mu[None, DH:]) * jnp.exp(-ls[None, DH:])
        g = -0.5 * (jnp.sum(r1 * r1, axis=1) + jnp.sum(r2 * r2, axis=1)
                    + 2.0 * jnp.sum(ls) + D * LOG2PI)
        lk_ref[pl.ds(k * KB + j, 1), :] = (g - jnp.sum(s, axis=1))[None, :]

    @pl.when(k == T // KB - 1)
    def _finish():
        pi = pi_ref[0, :]
        la = pi - (jnp.max(pi) + jnp.log(jnp.sum(jnp.exp(pi - jnp.max(pi)))))
        lp = lk_ref[:, :] + la[:, None]
        m = jnp.max(lp, axis=0)
        lse = m + jnp.log(jnp.sum(jnp.exp(lp - m[None, :]), axis=0))
        out_ref[0, :] = lse + lds_ref[0, :]


@jax.jit
def kernel(x, pi_logits, mus, log_sigmas, Ws1, bs1, Ws2, bs2, Wc1, bc1, Wc2, bc2):
    n = x.shape[0]
    Wc1r = Wc1.reshape(T, DH, H)
    bc1r = bc1.reshape(T, 1, H)
    Wc2r = Wc2.reshape(T, H, D)
    bc2r = bc2.reshape(T, 1, D)
    mus3 = mus.reshape(T, 1, D)
    ls3 = log_sigmas.reshape(T, 1, D)
    pi2 = pi_logits.reshape(1, T)

    grid = (n // TILE, T // KB)
    out = pl.pallas_call(
        _flow_kernel,
        grid=grid,
        in_specs=[
            pl.BlockSpec((TILE, D), lambda i, k: (i, 0)),
            pl.BlockSpec((1, T), lambda i, k: (0, 0)),
            pl.BlockSpec((KB, 1, D), lambda i, k: (k, 0, 0)),
            pl.BlockSpec((KB, 1, D), lambda i, k: (k, 0, 0)),
            pl.BlockSpec((NS, DH, H), lambda i, k: (0, 0, 0)),
            pl.BlockSpec((NS, H), lambda i, k: (0, 0)),
            pl.BlockSpec((NS, H, D), lambda i, k: (0, 0, 0)),
            pl.BlockSpec((NS, D), lambda i, k: (0, 0)),
            pl.BlockSpec((KB, DH, H), lambda i, k: (k, 0, 0)),
            pl.BlockSpec((KB, 1, H), lambda i, k: (k, 0, 0)),
            pl.BlockSpec((KB, H, D), lambda i, k: (k, 0, 0)),
            pl.BlockSpec((KB, 1, D), lambda i, k: (k, 0, 0)),
        ],
        out_specs=pl.BlockSpec((1, TILE), lambda i, k: (0, i)),
        out_shape=jax.ShapeDtypeStruct((1, n), jnp.float32),
        scratch_shapes=[
            pltpu.VMEM((TILE, DH), jnp.float32),
            pltpu.VMEM((TILE, DH), jnp.float32),
            pltpu.VMEM((1, TILE), jnp.float32),
            pltpu.VMEM((T, TILE), jnp.float32),
        ],
    )(x, pi2, mus3, ls3, Ws1, bs1, Ws2, bs2, Wc1r, bc1r, Wc2r, bc2r)
    return out[0]


# speedup vs baseline: 1.3166x; 1.0161x over previous
import math

import jax
import jax.numpy as jnp
from jax.experimental import pallas as pl
from jax.experimental.pallas import tpu as pltpu

T = 8
D = 1024
H = 512
NS = 2
N = 2048
TILE = 512
KB = 4
DH = D // 2
LOG2PI = math.log(2.0 * math.pi)


def _flow_kernel(x_ref, pi_ref, mus_ref, ls_ref,
                 Ws1_ref, bs1_ref, Ws2_ref, bs2_ref,
                 Wc1_ref, bc1_ref, Wc2_ref, bc2_ref,
                 out_ref,
                 A_ref, B_ref, lds_ref, lk_ref):
    k = pl.program_id(1)
    x1 = x_ref[:, :DH]
    x2 = x_ref[:, DH:]

    @pl.when(k == 0)
    def _shared():
        h1 = jnp.tanh(jnp.dot(x1, Ws1_ref[1], preferred_element_type=jnp.float32)
                      + bs1_ref[1][None, :])
        st1 = jnp.dot(h1, Ws2_ref[1], preferred_element_type=jnp.float32) + bs2_ref[1][None, :]
        s1 = jnp.tanh(st1[:, :DH])
        t1 = st1[:, DH:]
        h0 = jnp.tanh(jnp.dot(x1, Ws1_ref[0], preferred_element_type=jnp.float32)
                      + bs1_ref[0][None, :])
        st0 = jnp.dot(h0, Ws2_ref[0], preferred_element_type=jnp.float32) + bs2_ref[0][None, :]
        s0 = jnp.tanh(st0[:, :DH])
        t0 = st0[:, DH:]
        e0 = jnp.exp(-s0)
        A = jnp.exp(-(s0 + s1))
        A_ref[:, :] = A
        B_ref[:, :] = -(t1 * A + t0 * e0)
        lds_ref[0, :] = -jnp.sum(s0 + s1, axis=1)

    for j in range(KB):
        h = jnp.tanh(jnp.dot(x1, Wc1_ref[j], preferred_element_type=jnp.float32)
                     + bc1_ref[j])
        st = jnp.dot(h, Wc2_ref[j], preferred_element_type=jnp.float32) + bc2_ref[j]
        s = jnp.tanh(st[:, :DH])
        t = st[:, DH:]
        z2 = (x2 - t) * jnp.exp(-s) * A_ref[:, :] + B_ref[:, :]

        mu = mus_ref[j, 0]
        ls = ls_ref[j, 0]
        r1 = (x1 - mu[None, :DH]) * jnp.exp(-ls[None, :DH])
        r2 = (z2 - mu[None, DH:]) * jnp.exp(-ls[None, DH:])
        g = -0.5 * (jnp.sum(r1 * r1, axis=1) + jnp.sum(r2 * r2, axis=1)
                    + 2.0 * jnp.sum(ls) + D * LOG2PI)
        lk_ref[pl.ds(k * KB + j, 1), :] = (g - jnp.sum(s, axis=1))[None, :]

    @pl.when(k == T // KB - 1)
    def _finish():
        pi = pi_ref[0, :]
        la = pi - (jnp.max(pi) + jnp.log(jnp.sum(jnp.exp(pi - jnp.max(pi)))))
        lp = lk_ref[:, :] + la[:, None]
        m = jnp.max(lp, axis=0)
        lse = m + jnp.log(jnp.sum(jnp.exp(lp - m[None, :]), axis=0))
        out_ref[0, :] = lse + lds_ref[0, :]


@jax.jit
def kernel(x, pi_logits, mus, log_sigmas, Ws1, bs1, Ws2, bs2, Wc1, bc1, Wc2, bc2):
    n = x.shape[0]
    Wc1r = Wc1.reshape(T, DH, H)
    bc1r = bc1.reshape(T, 1, H)
    Wc2r = Wc2.reshape(T, H, D)
    bc2r = bc2.reshape(T, 1, D)
    mus3 = mus.reshape(T, 1, D)
    ls3 = log_sigmas.reshape(T, 1, D)
    pi2 = pi_logits.reshape(1, T)

    grid = (n // TILE, T // KB)
    out = pl.pallas_call(
        _flow_kernel,
        grid=grid,
        in_specs=[
            pl.BlockSpec((TILE, D), lambda i, k: (i, 0)),
            pl.BlockSpec((1, T), lambda i, k: (0, 0)),
            pl.BlockSpec((KB, 1, D), lambda i, k: (k, 0, 0)),
            pl.BlockSpec((KB, 1, D), lambda i, k: (k, 0, 0)),
            pl.BlockSpec((NS, DH, H), lambda i, k: (0, 0, 0)),
            pl.BlockSpec((NS, H), lambda i, k: (0, 0)),
            pl.BlockSpec((NS, H, D), lambda i, k: (0, 0, 0)),
            pl.BlockSpec((NS, D), lambda i, k: (0, 0)),
            pl.BlockSpec((KB, DH, H), lambda i, k: (k, 0, 0)),
            pl.BlockSpec((KB, 1, H), lambda i, k: (k, 0, 0)),
            pl.BlockSpec((KB, H, D), lambda i, k: (k, 0, 0)),
            pl.BlockSpec((KB, 1, D), lambda i, k: (k, 0, 0)),
        ],
        out_specs=pl.BlockSpec((1, TILE), lambda i, k: (0, i)),
        out_shape=jax.ShapeDtypeStruct((1, n), jnp.float32),
        scratch_shapes=[
            pltpu.VMEM((TILE, DH), jnp.float32),
            pltpu.VMEM((TILE, DH), jnp.float32),
            pltpu.VMEM((1, TILE), jnp.float32),
            pltpu.VMEM((T, TILE), jnp.float32),
        ],
    )(x, pi2, mus3, ls3, Ws1, bs1, Ws2, bs2, Wc1r, bc1r, Wc2r, bc2r)
    return out[0]


# speedup vs baseline: 1.4154x; 1.0750x over previous
import math

import jax
import jax.numpy as jnp
from jax.experimental import pallas as pl
from jax.experimental.pallas import tpu as pltpu

T = 8
D = 1024
H = 512
NS = 2
N = 2048
TILE = 512
KB = 8
DH = D // 2
LOG2PI = math.log(2.0 * math.pi)


def _flow_kernel(x_ref, pi_ref, mus_ref, ls_ref,
                 Ws1_ref, bs1_ref, Ws2_ref, bs2_ref,
                 Wc1_ref, bc1_ref, Wc2_ref, bc2_ref,
                 out_ref,
                 A_ref, B_ref, lds_ref, lk_ref):
    k = pl.program_id(1)
    x1 = x_ref[:, :DH]
    x2 = x_ref[:, DH:]

    @pl.when(k == 0)
    def _shared():
        h1 = jnp.tanh(jnp.dot(x1, Ws1_ref[1], preferred_element_type=jnp.float32)
                      + bs1_ref[1][None, :])
        st1 = jnp.dot(h1, Ws2_ref[1], preferred_element_type=jnp.float32) + bs2_ref[1][None, :]
        s1 = jnp.tanh(st1[:, :DH])
        t1 = st1[:, DH:]
        h0 = jnp.tanh(jnp.dot(x1, Ws1_ref[0], preferred_element_type=jnp.float32)
                      + bs1_ref[0][None, :])
        st0 = jnp.dot(h0, Ws2_ref[0], preferred_element_type=jnp.float32) + bs2_ref[0][None, :]
        s0 = jnp.tanh(st0[:, :DH])
        t0 = st0[:, DH:]
        e0 = jnp.exp(-s0)
        A = jnp.exp(-(s0 + s1))
        A_ref[:, :] = A
        B_ref[:, :] = -(t1 * A + t0 * e0)
        lds_ref[0, :] = -jnp.sum(s0 + s1, axis=1)

    for j in range(KB):
        h = jnp.tanh(jnp.dot(x1, Wc1_ref[j], preferred_element_type=jnp.float32)
                     + bc1_ref[j])
        st = jnp.dot(h, Wc2_ref[j], preferred_element_type=jnp.float32) + bc2_ref[j]
        s = jnp.tanh(st[:, :DH])
        t = st[:, DH:]
        z2 = (x2 - t) * jnp.exp(-s) * A_ref[:, :] + B_ref[:, :]

        mu = mus_ref[j, 0]
        ls = ls_ref[j, 0]
        r1 = (x1 - mu[None, :DH]) * jnp.exp(-ls[None, :DH])
        r2 = (z2 - mu[None, DH:]) * jnp.exp(-ls[None, DH:])
        g = -0.5 * (jnp.sum(r1 * r1, axis=1) + jnp.sum(r2 * r2, axis=1)
                    + 2.0 * jnp.sum(ls) + D * LOG2PI)
        lk_ref[pl.ds(k * KB + j, 1), :] = (g - jnp.sum(s, axis=1))[None, :]

    @pl.when(k == T // KB - 1)
    def _finish():
        pi = pi_ref[0, :]
        la = pi - (jnp.max(pi) + jnp.log(jnp.sum(jnp.exp(pi - jnp.max(pi)))))
        lp = lk_ref[:, :] + la[:, None]
        m = jnp.max(lp, axis=0)
        lse = m + jnp.log(jnp.sum(jnp.exp(lp - m[None, :]), axis=0))
        out_ref[0, :] = lse + lds_ref[0, :]


@jax.jit
def kernel(x, pi_logits, mus, log_sigmas, Ws1, bs1, Ws2, bs2, Wc1, bc1, Wc2, bc2):
    n = x.shape[0]
    Wc1r = Wc1.reshape(T, DH, H)
    bc1r = bc1.reshape(T, 1, H)
    Wc2r = Wc2.reshape(T, H, D)
    bc2r = bc2.reshape(T, 1, D)
    mus3 = mus.reshape(T, 1, D)
    ls3 = log_sigmas.reshape(T, 1, D)
    pi2 = pi_logits.reshape(1, T)

    grid = (n // TILE, T // KB)
    out = pl.pallas_call(
        _flow_kernel,
        grid=grid,
        in_specs=[
            pl.BlockSpec((TILE, D), lambda i, k: (i, 0)),
            pl.BlockSpec((1, T), lambda i, k: (0, 0)),
            pl.BlockSpec((KB, 1, D), lambda i, k: (k, 0, 0)),
            pl.BlockSpec((KB, 1, D), lambda i, k: (k, 0, 0)),
            pl.BlockSpec((NS, DH, H), lambda i, k: (0, 0, 0)),
            pl.BlockSpec((NS, H), lambda i, k: (0, 0)),
            pl.BlockSpec((NS, H, D), lambda i, k: (0, 0, 0)),
            pl.BlockSpec((NS, D), lambda i, k: (0, 0)),
            pl.BlockSpec((KB, DH, H), lambda i, k: (k, 0, 0)),
            pl.BlockSpec((KB, 1, H), lambda i, k: (k, 0, 0)),
            pl.BlockSpec((KB, H, D), lambda i, k: (k, 0, 0)),
            pl.BlockSpec((KB, 1, D), lambda i, k: (k, 0, 0)),
        ],
        out_specs=pl.BlockSpec((1, TILE), lambda i, k: (0, i)),
        out_shape=jax.ShapeDtypeStruct((1, n), jnp.float32),
        scratch_shapes=[
            pltpu.VMEM((TILE, DH), jnp.float32),
            pltpu.VMEM((TILE, DH), jnp.float32),
            pltpu.VMEM((1, TILE), jnp.float32),
            pltpu.VMEM((T, TILE), jnp.float32),
        ],
    )(x, pi2, mus3, ls3, Ws1, bs1, Ws2, bs2, Wc1r, bc1r, Wc2r, bc2r)
    return out[0]


# speedup vs baseline: 1.4683x; 1.0374x over previous
import math

import jax
import jax.numpy as jnp
from jax.experimental import pallas as pl
from jax.experimental.pallas import tpu as pltpu

T = 8
D = 1024
H = 512
NS = 2
N = 2048
TILE = 512
KB = 8
DH = D // 2
LOG2PI = math.log(2.0 * math.pi)


def _flow_kernel(x_ref, pi_ref, mus_ref, ls_ref,
                 Ws1_ref, bs1_ref, Ws2_ref, bs2_ref,
                 Wc1_ref, bc1_ref, Wc2_ref, bc2_ref,
                 out_ref,
                 A_ref, B_ref, lds_ref, lk_ref):
    k = pl.program_id(1)
    x1 = x_ref[:, :DH]
    x2 = x_ref[:, DH:]

    @pl.when(k == 0)
    def _shared():
        h1 = jnp.tanh(jnp.dot(x1, Ws1_ref[1], preferred_element_type=jnp.float32)
                      + bs1_ref[1][None, :])
        st1 = jnp.dot(h1, Ws2_ref[1], preferred_element_type=jnp.float32) + bs2_ref[1][None, :]
        s1 = jnp.tanh(st1[:, :DH])
        t1 = st1[:, DH:]
        h0 = jnp.tanh(jnp.dot(x1, Ws1_ref[0], preferred_element_type=jnp.float32)
                      + bs1_ref[0][None, :])
        st0 = jnp.dot(h0, Ws2_ref[0], preferred_element_type=jnp.float32) + bs2_ref[0][None, :]
        s0 = jnp.tanh(st0[:, :DH])
        t0 = st0[:, DH:]
        e0 = jnp.exp(-s0)
        A = jnp.exp(-(s0 + s1))
        A_ref[:, :] = A
        B_ref[:, :] = -(t1 * A + t0 * e0)
        lds_ref[0, :] = -jnp.sum(s0 + s1, axis=1)

    mu1a = mus_ref[:, 0, :DH]
    mu2a = mus_ref[:, 0, DH:]
    ls1a = ls_ref[:, 0, :DH]
    ls2a = ls_ref[:, 0, DH:]
    v1 = jnp.exp(-2.0 * ls1a)
    v2h = 0.5 * jnp.exp(-2.0 * ls2a)
    Wa = jnp.transpose(-0.5 * v1)
    Wb = jnp.transpose(mu1a * v1)
    G1 = (jnp.dot(x1 * x1, Wa, preferred_element_type=jnp.float32)
          + jnp.dot(x1, Wb, preferred_element_type=jnp.float32))
    cvec = (-0.5 * jnp.sum(mu1a * mu1a * v1, axis=1)
            - jnp.sum(ls1a, axis=1) - jnp.sum(ls2a, axis=1)
            - 0.5 * D * LOG2PI)

    A = A_ref[:, :]
    B = B_ref[:, :]
    for j in range(KB):
        h = jnp.tanh(jnp.dot(x1, Wc1_ref[j], preferred_element_type=jnp.float32)
                     + bc1_ref[j])
        st = jnp.dot(h, Wc2_ref[j], preferred_element_type=jnp.float32) + bc2_ref[j]
        s = jnp.tanh(st[:, :DH])
        t = st[:, DH:]
        zm = (x2 - t) * jnp.exp(-s) * A + (B - mu2a[j][None, :])
        expr = zm * zm * v2h[j][None, :] + s
        lk_ref[pl.ds(k * KB + j, 1), :] = -jnp.sum(expr, axis=1)[None, :]

    blk = pl.ds(k * KB, KB)
    lk_ref[blk, :] = lk_ref[blk, :] + jnp.transpose(G1) + cvec[:, None]

    @pl.when(k == T // KB - 1)
    def _finish():
        pi = pi_ref[0, :]
        la = pi - (jnp.max(pi) + jnp.log(jnp.sum(jnp.exp(pi - jnp.max(pi)))))
        lp = lk_ref[:, :] + la[:, None]
        m = jnp.max(lp, axis=0)
        lse = m + jnp.log(jnp.sum(jnp.exp(lp - m[None, :]), axis=0))
        out_ref[0, :] = lse + lds_ref[0, :]


@jax.jit
def kernel(x, pi_logits, mus, log_sigmas, Ws1, bs1, Ws2, bs2, Wc1, bc1, Wc2, bc2):
    n = x.shape[0]
    Wc1r = Wc1.reshape(T, DH, H)
    bc1r = bc1.reshape(T, 1, H)
    Wc2r = Wc2.reshape(T, H, D)
    bc2r = bc2.reshape(T, 1, D)
    mus3 = mus.reshape(T, 1, D)
    ls3 = log_sigmas.reshape(T, 1, D)
    pi2 = pi_logits.reshape(1, T)

    grid = (n // TILE, T // KB)
    out = pl.pallas_call(
        _flow_kernel,
        grid=grid,
        in_specs=[
            pl.BlockSpec((TILE, D), lambda i, k: (i, 0)),
            pl.BlockSpec((1, T), lambda i, k: (0, 0)),
            pl.BlockSpec((KB, 1, D), lambda i, k: (k, 0, 0)),
            pl.BlockSpec((KB, 1, D), lambda i, k: (k, 0, 0)),
            pl.BlockSpec((NS, DH, H), lambda i, k: (0, 0, 0)),
            pl.BlockSpec((NS, H), lambda i, k: (0, 0)),
            pl.BlockSpec((NS, H, D), lambda i, k: (0, 0, 0)),
            pl.BlockSpec((NS, D), lambda i, k: (0, 0)),
            pl.BlockSpec((KB, DH, H), lambda i, k: (k, 0, 0)),
            pl.BlockSpec((KB, 1, H), lambda i, k: (k, 0, 0)),
            pl.BlockSpec((KB, H, D), lambda i, k: (k, 0, 0)),
            pl.BlockSpec((KB, 1, D), lambda i, k: (k, 0, 0)),
        ],
        out_specs=pl.BlockSpec((1, TILE), lambda i, k: (0, i)),
        out_shape=jax.ShapeDtypeStruct((1, n), jnp.float32),
        scratch_shapes=[
            pltpu.VMEM((TILE, DH), jnp.float32),
            pltpu.VMEM((TILE, DH), jnp.float32),
            pltpu.VMEM((1, TILE), jnp.float32),
            pltpu.VMEM((T, TILE), jnp.float32),
        ],
    )(x, pi2, mus3, ls3, Ws1, bs1, Ws2, bs2, Wc1r, bc1r, Wc2r, bc2r)
    return out[0]


# speedup vs baseline: 1.5227x; 1.0371x over previous
import math

import jax
import jax.numpy as jnp
from jax.experimental import pallas as pl
from jax.experimental.pallas import tpu as pltpu

T = 8
D = 1024
H = 512
NS = 2
N = 2048
TILE = 512
KB = 8
DH = D // 2
LOG2PI = math.log(2.0 * math.pi)


def _flow_kernel(x_ref, pi_ref, mus_ref, ls_ref,
                 Ws1_ref, bs1_ref, Ws2_ref, bs2_ref,
                 Wc1_ref, bc1_ref, Wc2_ref, bc2_ref,
                 out_ref,
                 A_ref, B_ref, lds_ref, lk_ref):
    k = pl.program_id(1)
    x1 = x_ref[:, :DH]
    x2 = x_ref[:, DH:]

    @pl.when(k == 0)
    def _shared():
        h1 = jnp.tanh(jnp.dot(x1, Ws1_ref[1], preferred_element_type=jnp.float32)
                      + bs1_ref[1][None, :])
        st1 = jnp.dot(h1, Ws2_ref[1], preferred_element_type=jnp.float32) + bs2_ref[1][None, :]
        s1 = jnp.tanh(st1[:, :DH])
        t1 = st1[:, DH:]
        h0 = jnp.tanh(jnp.dot(x1, Ws1_ref[0], preferred_element_type=jnp.float32)
                      + bs1_ref[0][None, :])
        st0 = jnp.dot(h0, Ws2_ref[0], preferred_element_type=jnp.float32) + bs2_ref[0][None, :]
        s0 = jnp.tanh(st0[:, :DH])
        t0 = st0[:, DH:]
        e0 = jnp.exp(-s0)
        A = jnp.exp(-(s0 + s1))
        A_ref[:, :] = A
        B_ref[:, :] = -(t1 * A + t0 * e0)
        lds_ref[0, :] = -jnp.sum(s0 + s1, axis=1)

    mu1a = mus_ref[:, 0, :DH]
    mu2a = mus_ref[:, 0, DH:]
    ls1a = ls_ref[:, 0, :DH]
    ls2a = ls_ref[:, 0, DH:]
    v1 = jnp.exp(-2.0 * ls1a)
    v2h = 0.5 * jnp.exp(-2.0 * ls2a)
    Wa = jnp.transpose(-0.5 * v1)
    Wb = jnp.transpose(mu1a * v1)
    G1 = (jnp.dot(x1 * x1, Wa, preferred_element_type=jnp.float32)
          + jnp.dot(x1, Wb, preferred_element_type=jnp.float32))
    cvec = (-0.5 * jnp.sum(mu1a * mu1a * v1, axis=1)
            - jnp.sum(ls1a, axis=1) - jnp.sum(ls2a, axis=1)
            - 0.5 * D * LOG2PI)

    A = A_ref[:, :]
    B = B_ref[:, :]
    ones_dh = jnp.ones((DH, 1), dtype=jnp.float32)
    for j in range(KB):
        h = jnp.tanh(jnp.dot(x1, Wc1_ref[j], preferred_element_type=jnp.float32)
                     + bc1_ref[j])
        st = jnp.dot(h, Wc2_ref[j], preferred_element_type=jnp.float32) + bc2_ref[j]
        s = jnp.tanh(st[:, :DH])
        t = st[:, DH:]
        zm = (x2 - t) * jnp.exp(-s) * A + (B - mu2a[j][None, :])
        expr = zm * zm * v2h[j][None, :] + s
        red = jnp.dot(expr, ones_dh, preferred_element_type=jnp.float32)
        lk_ref[pl.ds(k * KB + j, 1), :] = -red.reshape(1, TILE)

    blk = pl.ds(k * KB, KB)
    lk_ref[blk, :] = lk_ref[blk, :] + jnp.transpose(G1) + cvec[:, None]

    @pl.when(k == T // KB - 1)
    def _finish():
        pi = pi_ref[0, :]
        la = pi - (jnp.max(pi) + jnp.log(jnp.sum(jnp.exp(pi - jnp.max(pi)))))
        lp = lk_ref[:, :] + la[:, None]
        m = jnp.max(lp, axis=0)
        lse = m + jnp.log(jnp.sum(jnp.exp(lp - m[None, :]), axis=0))
        out_ref[0, :] = lse + lds_ref[0, :]


@jax.jit
def kernel(x, pi_logits, mus, log_sigmas, Ws1, bs1, Ws2, bs2, Wc1, bc1, Wc2, bc2):
    n = x.shape[0]
    Wc1r = Wc1.reshape(T, DH, H)
    bc1r = bc1.reshape(T, 1, H)
    Wc2r = Wc2.reshape(T, H, D)
    bc2r = bc2.reshape(T, 1, D)
    mus3 = mus.reshape(T, 1, D)
    ls3 = log_sigmas.reshape(T, 1, D)
    pi2 = pi_logits.reshape(1, T)

    grid = (n // TILE, T // KB)
    out = pl.pallas_call(
        _flow_kernel,
        grid=grid,
        in_specs=[
            pl.BlockSpec((TILE, D), lambda i, k: (i, 0)),
            pl.BlockSpec((1, T), lambda i, k: (0, 0)),
            pl.BlockSpec((KB, 1, D), lambda i, k: (k, 0, 0)),
            pl.BlockSpec((KB, 1, D), lambda i, k: (k, 0, 0)),
            pl.BlockSpec((NS, DH, H), lambda i, k: (0, 0, 0)),
            pl.BlockSpec((NS, H), lambda i, k: (0, 0)),
            pl.BlockSpec((NS, H, D), lambda i, k: (0, 0, 0)),
            pl.BlockSpec((NS, D), lambda i, k: (0, 0)),
            pl.BlockSpec((KB, DH, H), lambda i, k: (k, 0, 0)),
            pl.BlockSpec((KB, 1, H), lambda i, k: (k, 0, 0)),
            pl.BlockSpec((KB, H, D), lambda i, k: (k, 0, 0)),
            pl.BlockSpec((KB, 1, D), lambda i, k: (k, 0, 0)),
        ],
        out_specs=pl.BlockSpec((1, TILE), lambda i, k: (0, i)),
        out_shape=jax.ShapeDtypeStruct((1, n), jnp.float32),
        scratch_shapes=[
            pltpu.VMEM((TILE, DH), jnp.float32),
            pltpu.VMEM((TILE, DH), jnp.float32),
            pltpu.VMEM((1, TILE), jnp.float32),
            pltpu.VMEM((T, TILE), jnp.float32),
        ],
    )(x, pi2, mus3, ls3, Ws1, bs1, Ws2, bs2, Wc1r, bc1r, Wc2r, bc2r)
    return out[0]


# speedup vs baseline: 1.5344x; 1.0077x over previous
import math

import jax
import jax.numpy as jnp
from jax.experimental import pallas as pl

T = 8
D = 1024
H = 512
NS = 2
TILE = 512
DH = D // 2
LOG2PI = math.log(2.0 * math.pi)


def _flow_kernel(x_ref, pi_ref, mus_ref, ls_ref,
                 Ws1_ref, bs1_ref, Ws2_ref, bs2_ref,
                 Wc1_ref, bc1_ref, Wc2_ref, bc2_ref,
                 out_ref):
    x1 = x_ref[:, :DH]
    x2 = x_ref[:, DH:]
    ones_dh = jnp.ones((DH, 1), dtype=jnp.float32)

    h1 = jnp.tanh(jnp.dot(x1, Ws1_ref[1], preferred_element_type=jnp.float32)
                  + bs1_ref[1][None, :])
    st1 = jnp.dot(h1, Ws2_ref[1], preferred_element_type=jnp.float32) + bs2_ref[1][None, :]
    s1 = jnp.tanh(st1[:, :DH])
    t1 = st1[:, DH:]
    h0 = jnp.tanh(jnp.dot(x1, Ws1_ref[0], preferred_element_type=jnp.float32)
                  + bs1_ref[0][None, :])
    st0 = jnp.dot(h0, Ws2_ref[0], preferred_element_type=jnp.float32) + bs2_ref[0][None, :]
    s0 = jnp.tanh(st0[:, :DH])
    t0 = st0[:, DH:]
    e0 = jnp.exp(-s0)
    A = jnp.exp(-(s0 + s1))
    B = -(t1 * A + t0 * e0)
    lds = jnp.dot(s0 + s1, ones_dh, preferred_element_type=jnp.float32)

    mu1a = mus_ref[:, 0, :DH]
    mu2a = mus_ref[:, 0, DH:]
    ls1a = ls_ref[:, 0, :DH]
    ls2a = ls_ref[:, 0, DH:]
    v1 = jnp.exp(-2.0 * ls1a)
    v2h = 0.5 * jnp.exp(-2.0 * ls2a)
    Wa = jnp.transpose(-0.5 * v1)
    Wb = jnp.transpose(mu1a * v1)
    G1 = (jnp.dot(x1 * x1, Wa, preferred_element_type=jnp.float32)
          + jnp.dot(x1, Wb, preferred_element_type=jnp.float32))
    cvec = (-0.5 * jnp.sum(mu1a * mu1a * v1, axis=1)
            - jnp.sum(ls1a, axis=1) - jnp.sum(ls2a, axis=1)
            - 0.5 * D * LOG2PI)
    pi = pi_ref[0, :]
    la = pi - (jnp.max(pi) + jnp.log(jnp.sum(jnp.exp(pi - jnp.max(pi)))))

    reds = []
    for j in range(T):
        h = jnp.tanh(jnp.dot(x1, Wc1_ref[j], preferred_element_type=jnp.float32)
                     + bc1_ref[j])
        st = jnp.dot(h, Wc2_ref[j], preferred_element_type=jnp.float32) + bc2_ref[j]
        s = jnp.tanh(st[:, :DH])
        t = st[:, DH:]
        zm = (x2 - t) * jnp.exp(-s) * A + (B - mu2a[j][None, :])
        expr = zm * zm * v2h[j][None, :] + s
        reds.append(jnp.dot(expr, ones_dh, preferred_element_type=jnp.float32))

    lp = G1 - jnp.concatenate(reds, axis=1) + (la + cvec)[None, :]
    m = jnp.max(lp, axis=1, keepdims=True)
    lse = m + jnp.log(jnp.sum(jnp.exp(lp - m), axis=1, keepdims=True))
    out_ref[:, :] = lse - lds


@jax.jit
def kernel(x, pi_logits, mus, log_sigmas, Ws1, bs1, Ws2, bs2, Wc1, bc1, Wc2, bc2):
    n = x.shape[0]
    Wc1r = Wc1.reshape(T, DH, H)
    bc1r = bc1.reshape(T, 1, H)
    Wc2r = Wc2.reshape(T, H, D)
    bc2r = bc2.reshape(T, 1, D)
    mus3 = mus.reshape(T, 1, D)
    ls3 = log_sigmas.reshape(T, 1, D)
    pi2 = pi_logits.reshape(1, T)

    grid = (n // TILE,)
    out = pl.pallas_call(
        _flow_kernel,
        grid=grid,
        in_specs=[
            pl.BlockSpec((TILE, D), lambda i: (i, 0)),
            pl.BlockSpec((1, T), lambda i: (0, 0)),
            pl.BlockSpec((T, 1, D), lambda i: (0, 0, 0)),
            pl.BlockSpec((T, 1, D), lambda i: (0, 0, 0)),
            pl.BlockSpec((NS, DH, H), lambda i: (0, 0, 0)),
            pl.BlockSpec((NS, H), lambda i: (0, 0)),
            pl.BlockSpec((NS, H, D), lambda i: (0, 0, 0)),
            pl.BlockSpec((NS, D), lambda i: (0, 0)),
            pl.BlockSpec((T, DH, H), lambda i: (0, 0, 0)),
            pl.BlockSpec((T, 1, H), lambda i: (0, 0, 0)),
            pl.BlockSpec((T, H, D), lambda i: (0, 0, 0)),
            pl.BlockSpec((T, 1, D), lambda i: (0, 0, 0)),
        ],
        out_specs=pl.BlockSpec((TILE, 1), lambda i: (i, 0)),
        out_shape=jax.ShapeDtypeStruct((n, 1), jnp.float32),
    )(x, pi2, mus3, ls3, Ws1, bs1, Ws2, bs2, Wc1r, bc1r, Wc2r, bc2r)
    return out[:, 0]
